# baseline (speedup 1.0000x reference)
import jax
import jax.numpy as jnp
from jax import lax
from jax.experimental import pallas as pl
from jax.experimental.pallas import tpu as pltpu
from jax.experimental.pallas import tpu_sc as plsc

_N_CODES = 8192
_CODE_DIM = 64
_N_VECS = 16384
_R = 512

_NC, _NS = 2, 16
_NW = _NC * _NS
_BPW = _N_VECS // _NW
_KCH = _BPW // 128
_D_PAD = 128


def _dist_argmin_kernel(x_ref, v2_ref, cb_ref, c2_ref, out_ref):
    s = lax.dot_general(
        x_ref[...], cb_ref[...], (((1,), (1,)), ((), ())),
        preferred_element_type=jnp.float32)
    d = (v2_ref[...] - s) + c2_ref[...]
    m = jnp.min(d, axis=1, keepdims=True)
    ids = lax.broadcasted_iota(jnp.int32, d.shape, 1)
    idx = jnp.min(jnp.where(d == m, ids, _N_CODES), axis=1).astype(jnp.int32)
    out_ref[...] = idx.reshape(1, 1, _R)


def _compute_indices(x, v2, codebook, c2):
    idx3 = pl.pallas_call(
        _dist_argmin_kernel,
        grid=(_N_VECS // _R,),
        in_specs=[
            pl.BlockSpec((_R, _CODE_DIM), lambda i: (i, 0)),
            pl.BlockSpec((_R, 1), lambda i: (i, 0)),
            pl.BlockSpec((_N_CODES, _CODE_DIM), lambda i: (0, 0)),
            pl.BlockSpec((1, _N_CODES), lambda i: (0, 0)),
        ],
        out_specs=pl.BlockSpec((1, 1, _R), lambda i: (i, 0, 0)),
        out_shape=jax.ShapeDtypeStruct((_N_VECS // _R, 1, _R), jnp.int32),
    )(x, v2, codebook, c2)
    return idx3.reshape(_NW, _KCH, 128)


def _gather_body(idx_hbm, table_hbm, out_hbm, idx_v, rows_v, sem):
    wid = lax.axis_index("s") * _NC + lax.axis_index("c")
    pltpu.sync_copy(idx_hbm.at[wid], idx_v)
    copies = [
        pltpu.async_copy(table_hbm.at[idx_v.at[j]], rows_v.at[j], sem)
        for j in range(_KCH)
    ]
    for c in copies:
        c.wait()
    pltpu.sync_copy(rows_v, out_hbm.at[wid])


_gather = pl.kernel(
    _gather_body,
    mesh=plsc.VectorSubcoreMesh(core_axis_name="c", subcore_axis_name="s"),
    out_type=jax.ShapeDtypeStruct((_NW, _KCH, 128, _D_PAD), jnp.float32),
    scratch_types=[
        pltpu.VMEM((_KCH, 128), jnp.int32),
        pltpu.VMEM((_KCH, 128, _D_PAD), jnp.float32),
        pltpu.SemaphoreType.DMA,
    ],
)


def kernel(inputs, codebook):
    input_shape = inputs.shape
    x = jnp.transpose(inputs, (0, 2, 3, 1)).reshape(-1, _CODE_DIM)
    v2 = jnp.sum(x ** 2, axis=1, keepdims=True)
    c2 = jnp.sum(codebook ** 2, axis=1)[None, :]
    idx = _compute_indices(x * 2.0, v2, codebook, c2)
    table = codebook.astype(jnp.bfloat16).astype(jnp.float32)
    table = jnp.pad(table, ((0, 0), (0, _D_PAD - _CODE_DIM)))
    quantized = _gather(idx, table)
    quantized = quantized.reshape(_N_VECS, _D_PAD)[:, :_CODE_DIM]
    return quantized.reshape(input_shape)


# speedup vs baseline: 1.0201x; 1.0201x over previous
import jax
import jax.numpy as jnp
from jax import lax
from jax.experimental import pallas as pl
from jax.experimental.pallas import tpu as pltpu
from jax.experimental.pallas import tpu_sc as plsc

_N_CODES = 8192
_CODE_DIM = 64
_N_VECS = 16384
_R = 512

_NC, _NS = 2, 16
_NW = _NC * _NS
_BPW = _N_VECS // _NW
_KCH = _BPW // 128
_D_PAD = 128


def _dist_argmin_kernel(x_ref, v2_ref, cb_ref, c2_ref, out_ref):
    s = lax.dot_general(
        x_ref[...] * 2.0, cb_ref[...], (((1,), (1,)), ((), ())),
        preferred_element_type=jnp.float32)
    d = (v2_ref[...] - s) + c2_ref[...]
    m = jnp.min(d, axis=1, keepdims=True)
    ids = lax.broadcasted_iota(jnp.int32, d.shape, 1)
    idx = jnp.min(jnp.where(d == m, ids, _N_CODES), axis=1).astype(jnp.int32)
    out_ref[...] = idx.reshape(1, 1, _R)


def _compute_indices(x, v2, codebook, c2):
    idx3 = pl.pallas_call(
        _dist_argmin_kernel,
        grid=(_N_VECS // _R,),
        in_specs=[
            pl.BlockSpec((_R, _CODE_DIM), lambda i: (i, 0)),
            pl.BlockSpec((_R, 1), lambda i: (i, 0)),
            pl.BlockSpec((_N_CODES, _CODE_DIM), lambda i: (0, 0)),
            pl.BlockSpec((1, _N_CODES), lambda i: (0, 0)),
        ],
        out_specs=pl.BlockSpec((1, 1, _R), lambda i: (i, 0, 0)),
        out_shape=jax.ShapeDtypeStruct((_N_VECS // _R, 1, _R), jnp.int32),
    )(x, v2, codebook, c2)
    return idx3.reshape(_NW, _KCH, 128)


def _gather_body(idx_hbm, table_hbm, out_hbm, idx_v, rows_v, sem):
    wid = lax.axis_index("s") * _NC + lax.axis_index("c")
    pltpu.sync_copy(idx_hbm.at[wid], idx_v)
    copies = [
        pltpu.async_copy(table_hbm.at[idx_v.at[j]], rows_v.at[j], sem)
        for j in range(_KCH)
    ]
    for c in copies:
        c.wait()
    pltpu.sync_copy(rows_v, out_hbm.at[wid])


_gather = pl.kernel(
    _gather_body,
    mesh=plsc.VectorSubcoreMesh(core_axis_name="c", subcore_axis_name="s"),
    out_type=jax.ShapeDtypeStruct((_NW, _KCH, 128, _D_PAD), jnp.float32),
    scratch_types=[
        pltpu.VMEM((_KCH, 128), jnp.int32),
        pltpu.VMEM((_KCH, 128, _D_PAD), jnp.float32),
        pltpu.SemaphoreType.DMA,
    ],
)


def kernel(inputs, codebook):
    input_shape = inputs.shape
    x = jnp.transpose(inputs, (0, 2, 3, 1)).reshape(-1, _CODE_DIM)
    v2 = jnp.sum(x ** 2, axis=1, keepdims=True)
    c2 = jnp.sum(codebook ** 2, axis=1)[None, :]
    idx = _compute_indices(x, v2, codebook, c2)
    table = codebook.astype(jnp.bfloat16).astype(jnp.float32)
    table = jnp.pad(table, ((0, 0), (0, _D_PAD - _CODE_DIM)))
    quantized = _gather(idx, table)
    quantized = quantized.reshape(_N_VECS, _D_PAD)[:, :_CODE_DIM]
    return quantized.reshape(input_shape)


# speedup vs baseline: 1.0907x; 1.0692x over previous
import jax
import jax.numpy as jnp
from jax import lax
from jax.experimental import pallas as pl
from jax.experimental.pallas import tpu as pltpu
from jax.experimental.pallas import tpu_sc as plsc

_N_CODES = 8192
_CODE_DIM = 64
_N_VECS = 16384
_R = 1024

_NC, _NS = 2, 16
_NW = _NC * _NS
_BPW = _N_VECS // _NW
_KCH = _BPW // 128
_D_PAD = 128


def _dist_argmin_kernel(x_ref, v2_ref, cb_ref, c2_ref, out_ref):
    s = lax.dot_general(
        x_ref[...], cb_ref[...], (((1,), (1,)), ((), ())),
        preferred_element_type=jnp.float32)
    d = (v2_ref[...] - 2.0 * s) + c2_ref[...]
    m = jnp.min(d, axis=1, keepdims=True)
    ids = lax.broadcasted_iota(jnp.int32, d.shape, 1)
    idx = jnp.min(jnp.where(d == m, ids, _N_CODES), axis=1).astype(jnp.int32)
    out_ref[...] = idx.reshape(1, 1, _R)


def _compute_indices(x, v2, codebook, c2):
    idx3 = pl.pallas_call(
        _dist_argmin_kernel,
        grid=(_N_VECS // _R,),
        in_specs=[
            pl.BlockSpec((_R, _CODE_DIM), lambda i: (i, 0)),
            pl.BlockSpec((_R, 1), lambda i: (i, 0)),
            pl.BlockSpec((_N_CODES, _CODE_DIM), lambda i: (0, 0)),
            pl.BlockSpec((1, _N_CODES), lambda i: (0, 0)),
        ],
        out_specs=pl.BlockSpec((1, 1, _R), lambda i: (i, 0, 0)),
        out_shape=jax.ShapeDtypeStruct((_N_VECS // _R, 1, _R), jnp.int32),
    )(x, v2, codebook, c2)
    return idx3.reshape(_NW, _KCH, 128)


def _gather_body(idx_hbm, table_hbm, out_hbm, idx_v, rows_v, sem):
    wid = lax.axis_index("s") * _NC + lax.axis_index("c")
    pltpu.sync_copy(idx_hbm.at[wid], idx_v)
    copies = [
        pltpu.async_copy(table_hbm.at[idx_v.at[j]], rows_v.at[j], sem)
        for j in range(_KCH)
    ]
    for c in copies:
        c.wait()
    pltpu.sync_copy(rows_v, out_hbm.at[wid])


_gather = pl.kernel(
    _gather_body,
    mesh=plsc.VectorSubcoreMesh(core_axis_name="c", subcore_axis_name="s"),
    out_type=jax.ShapeDtypeStruct((_NW, _KCH, 128, _D_PAD), jnp.float32),
    scratch_types=[
        pltpu.VMEM((_KCH, 128), jnp.int32),
        pltpu.VMEM((_KCH, 128, _D_PAD), jnp.float32),
        pltpu.SemaphoreType.DMA,
    ],
)


def kernel(inputs, codebook):
    input_shape = inputs.shape
    x = jnp.transpose(inputs, (0, 2, 3, 1)).reshape(-1, _CODE_DIM)
    v2 = jnp.sum(x ** 2, axis=1, keepdims=True)
    c2 = jnp.sum(codebook ** 2, axis=1)[None, :]
    idx = _compute_indices(x, v2, codebook, c2)
    table = codebook.astype(jnp.bfloat16).astype(jnp.float32)
    table = jnp.pad(table, ((0, 0), (0, _D_PAD - _CODE_DIM)))
    quantized = _gather(idx, table)
    quantized = quantized.reshape(_N_VECS, _D_PAD)[:, :_CODE_DIM]
    return quantized.reshape(input_shape)


# speedup vs baseline: 1.1950x; 1.0957x over previous
import jax
import jax.numpy as jnp
from jax import lax
from jax.experimental import pallas as pl
from jax.experimental.pallas import tpu as pltpu
from jax.experimental.pallas import tpu_sc as plsc

_N_CODES = 8192
_CODE_DIM = 64
_N_VECS = 16384
_R = 1024

_NC, _NS = 2, 16
_NW = _NC * _NS
_BPW = _N_VECS // _NW
_KCH = _BPW // 128
_D_PAD = 128


def _dist_argmin_kernel(x_ref, v2_ref, cb_ref, c2_ref, out_ref):
    s = lax.dot_general(
        x_ref[...], cb_ref[...], (((1,), (1,)), ((), ())),
        preferred_element_type=jnp.float32)
    d = (v2_ref[...] - 2.0 * s) + c2_ref[...]
    m = jnp.min(d, axis=1, keepdims=True)
    ids = lax.broadcasted_iota(jnp.int32, d.shape, 1).astype(jnp.float32)
    idx_f = jnp.min(jnp.where(d == m, ids, jnp.float32(_N_CODES)), axis=1)
    out_ref[...] = idx_f.astype(jnp.int32).reshape(1, 1, _R)


def _compute_indices(x, v2, codebook, c2):
    idx3 = pl.pallas_call(
        _dist_argmin_kernel,
        grid=(_N_VECS // _R,),
        in_specs=[
            pl.BlockSpec((_R, _CODE_DIM), lambda i: (i, 0)),
            pl.BlockSpec((_R, 1), lambda i: (i, 0)),
            pl.BlockSpec((_N_CODES, _CODE_DIM), lambda i: (0, 0)),
            pl.BlockSpec((1, _N_CODES), lambda i: (0, 0)),
        ],
        out_specs=pl.BlockSpec((1, 1, _R), lambda i: (i, 0, 0)),
        out_shape=jax.ShapeDtypeStruct((_N_VECS // _R, 1, _R), jnp.int32),
    )(x, v2, codebook, c2)
    return idx3.reshape(_NW, _KCH, 128)


def _gather_body(idx_hbm, table_hbm, out_hbm, idx_v, rows_v, sem):
    wid = lax.axis_index("s") * _NC + lax.axis_index("c")
    pltpu.sync_copy(idx_hbm.at[wid], idx_v)
    copies = [
        pltpu.async_copy(table_hbm.at[idx_v.at[j]], rows_v.at[j], sem)
        for j in range(_KCH)
    ]
    for c in copies:
        c.wait()
    pltpu.sync_copy(rows_v, out_hbm.at[wid])


_gather = pl.kernel(
    _gather_body,
    mesh=plsc.VectorSubcoreMesh(core_axis_name="c", subcore_axis_name="s"),
    out_type=jax.ShapeDtypeStruct((_NW, _KCH, 128, _D_PAD), jnp.float32),
    scratch_types=[
        pltpu.VMEM((_KCH, 128), jnp.int32),
        pltpu.VMEM((_KCH, 128, _D_PAD), jnp.float32),
        pltpu.SemaphoreType.DMA,
    ],
)


def kernel(inputs, codebook):
    input_shape = inputs.shape
    x = jnp.transpose(inputs, (0, 2, 3, 1)).reshape(-1, _CODE_DIM)
    v2 = jnp.sum(x ** 2, axis=1, keepdims=True)
    c2 = jnp.sum(codebook ** 2, axis=1)[None, :]
    idx = _compute_indices(x, v2, codebook, c2)
    table = codebook.astype(jnp.bfloat16).astype(jnp.float32)
    table = jnp.pad(table, ((0, 0), (0, _D_PAD - _CODE_DIM)))
    quantized = _gather(idx, table)
    quantized = quantized.reshape(_N_VECS, _D_PAD)[:, :_CODE_DIM]
    return quantized.reshape(input_shape)
